# Initial kernel scaffold; baseline (speedup 1.0000x reference)
import jax
import jax.numpy as jnp
from jax.experimental import pallas as pl


def kernel(input_ids, embedding_weight):
    raise NotImplementedError("write your pallas kernel here")


# speedup vs baseline: 2.9578x; 2.9578x over previous
import jax
import jax.numpy as jnp
from jax import lax
from jax.experimental import pallas as pl
from jax.experimental.pallas import tpu as pltpu
from jax.experimental.pallas import tpu_sc as plsc

DIM = 128
NUM_CORES = 2
NUM_SUBCORES = 16
NUM_WORKERS = NUM_CORES * NUM_SUBCORES
GROUP = 128


def _emb_body(idx_hbm, table_hbm, out_hbm, idx_v, rows_v, gsem):
    n_groups = idx_hbm.shape[1]
    wid = lax.axis_index("s") * NUM_CORES + lax.axis_index("c")
    pltpu.sync_copy(idx_hbm.at[wid], idx_v)

    def body(j, carry):
        pltpu.async_copy(table_hbm.at[idx_v.at[j]], rows_v, gsem).wait()
        pltpu.sync_copy(rows_v, out_hbm.at[wid, j])
        return carry

    lax.fori_loop(0, n_groups, body, 0)


def kernel(input_ids, embedding_weight):
    batch, seq = input_ids.shape
    total = batch * seq
    n_groups = total // (NUM_WORKERS * GROUP)
    assert n_groups * NUM_WORKERS * GROUP == total

    idx = input_ids.reshape(NUM_WORKERS, n_groups, GROUP).astype(jnp.int32)
    mesh = plsc.VectorSubcoreMesh(core_axis_name="c", subcore_axis_name="s")
    out = pl.kernel(
        _emb_body,
        out_type=jax.ShapeDtypeStruct((NUM_WORKERS, n_groups, GROUP, DIM),
                                      jnp.float32),
        mesh=mesh,
        scratch_types=[
            pltpu.VMEM((n_groups, GROUP), jnp.int32),
            pltpu.VMEM((GROUP, DIM), jnp.float32),
            pltpu.SemaphoreType.DMA,
        ],
    )(idx, embedding_weight)
    return out.reshape(batch, seq, DIM)


# speedup vs baseline: 3.3144x; 1.1205x over previous
import jax
import jax.numpy as jnp
from jax import lax
from jax.experimental import pallas as pl
from jax.experimental.pallas import tpu as pltpu
from jax.experimental.pallas import tpu_sc as plsc

DIM = 128
NUM_CORES = 2
NUM_SUBCORES = 16
NUM_WORKERS = NUM_CORES * NUM_SUBCORES
GROUP = 128
NBUF = 5


def _emb_body(idx_hbm, table_hbm, out_hbm, idx_v, rows_v, gsem, osem):
    n_groups = idx_hbm.shape[1]
    n_rot = n_groups // NBUF
    wid = lax.axis_index("s") * NUM_CORES + lax.axis_index("c")
    pltpu.sync_copy(idx_hbm.at[wid], idx_v)

    def gstart(g, b):
        pltpu.async_copy(table_hbm.at[idx_v.at[g]], rows_v.at[b], gsem.at[b])

    def gwait(g, b):
        pltpu.make_async_copy(table_hbm.at[idx_v.at[g]], rows_v.at[b],
                              gsem.at[b]).wait()

    def wstart(g, b):
        pltpu.async_copy(rows_v.at[b], out_hbm.at[wid, g], osem.at[b])

    def wwait(g, b):
        pltpu.make_async_copy(rows_v.at[b], out_hbm.at[wid, g],
                              osem.at[b]).wait()

    for b in range(NBUF):
        gstart(b, b)

    def body(r, carry):
        g0 = r * NBUF
        for b in range(NBUF):
            gwait(g0 + b, b)
            wstart(g0 + b, b)
        for b in range(NBUF):
            wwait(g0 + b, b)
            gstart(g0 + NBUF + b, b)
        return carry

    lax.fori_loop(0, n_rot - 1, body, 0)

    g0 = (n_rot - 1) * NBUF
    for b in range(NBUF):
        gwait(g0 + b, b)
        wstart(g0 + b, b)
    for b in range(NBUF):
        wwait(g0 + b, b)


def kernel(input_ids, embedding_weight):
    batch, seq = input_ids.shape
    total = batch * seq
    n_groups = total // (NUM_WORKERS * GROUP)
    assert n_groups * NUM_WORKERS * GROUP == total
    assert n_groups % NBUF == 0

    idx = input_ids.reshape(NUM_WORKERS, n_groups, GROUP).astype(jnp.int32)
    mesh = plsc.VectorSubcoreMesh(core_axis_name="c", subcore_axis_name="s")
    out = pl.kernel(
        _emb_body,
        out_type=jax.ShapeDtypeStruct((NUM_WORKERS, n_groups, GROUP, DIM),
                                      jnp.float32),
        mesh=mesh,
        scratch_types=[
            pltpu.VMEM((n_groups, GROUP), jnp.int32),
            pltpu.VMEM((NBUF, GROUP, DIM), jnp.float32),
            pltpu.SemaphoreType.DMA((NBUF,)),
            pltpu.SemaphoreType.DMA((NBUF,)),
        ],
    )(idx, embedding_weight)
    return out.reshape(batch, seq, DIM)


# speedup vs baseline: 5.9070x; 1.7823x over previous
import jax
import jax.numpy as jnp
from jax import lax
from jax.experimental import pallas as pl
from jax.experimental.pallas import tpu as pltpu
from jax.experimental.pallas import tpu_sc as plsc

DIM = 128
NUM_CORES = 2
NUM_SUBCORES = 16
NUM_WORKERS = NUM_CORES * NUM_SUBCORES
NBUF = 8


def _emb_body(idx_hbm, table_hbm, out_hbm, idx_v, rows_v, gsem, osem):
    rows_per_w = idx_hbm.shape[1]
    seq = out_hbm.shape[1]
    n_rot = rows_per_w // NBUF
    wid = lax.axis_index("s") * NUM_CORES + lax.axis_index("c")
    row0 = wid * rows_per_w
    pltpu.sync_copy(idx_hbm.at[wid], idx_v)

    def gstart(g, b):
        pltpu.async_copy(table_hbm.at[idx_v.at[g, pl.ds(0, seq)]],
                         rows_v.at[b], gsem.at[b])

    def gwait(g, b):
        pltpu.make_async_copy(table_hbm.at[idx_v.at[g, pl.ds(0, seq)]],
                              rows_v.at[b], gsem.at[b]).wait()

    def wstart(g, b):
        pltpu.async_copy(rows_v.at[b], out_hbm.at[row0 + g], osem.at[b])

    def wwait(g, b):
        pltpu.make_async_copy(rows_v.at[b], out_hbm.at[row0 + g],
                              osem.at[b]).wait()

    for b in range(NBUF):
        gstart(b, b)

    def body(r, carry):
        g0 = r * NBUF
        for b in range(NBUF):
            gwait(g0 + b, b)
            wstart(g0 + b, b)
        for b in range(NBUF):
            wwait(g0 + b, b)
            gstart(g0 + NBUF + b, b)
        return carry

    lax.fori_loop(0, n_rot - 1, body, 0)

    g0 = (n_rot - 1) * NBUF
    for b in range(NBUF):
        gwait(g0 + b, b)
        wstart(g0 + b, b)
    for b in range(NBUF):
        wwait(g0 + b, b)


def kernel(input_ids, embedding_weight):
    batch, seq = input_ids.shape
    rows_per_w = batch // NUM_WORKERS
    assert rows_per_w * NUM_WORKERS == batch
    assert rows_per_w % NBUF == 0

    seq_pad = (seq + 7) // 8 * 8
    idx = input_ids.astype(jnp.int32)
    if seq_pad != seq:
        idx = jnp.pad(idx, ((0, 0), (0, seq_pad - seq)))
    idx = idx.reshape(NUM_WORKERS, rows_per_w, seq_pad)
    mesh = plsc.VectorSubcoreMesh(core_axis_name="c", subcore_axis_name="s")
    out = pl.kernel(
        _emb_body,
        out_type=jax.ShapeDtypeStruct((batch, seq, DIM), jnp.float32),
        mesh=mesh,
        scratch_types=[
            pltpu.VMEM((rows_per_w, seq_pad), jnp.int32),
            pltpu.VMEM((NBUF, seq, DIM), jnp.float32),
            pltpu.SemaphoreType.DMA((NBUF,)),
            pltpu.SemaphoreType.DMA((NBUF,)),
        ],
    )(idx, embedding_weight)
    return out


# speedup vs baseline: 10.3597x; 1.7538x over previous
import jax
import jax.numpy as jnp
from jax import lax
from jax.experimental import pallas as pl
from jax.experimental.pallas import tpu as pltpu
from jax.experimental.pallas import tpu_sc as plsc

DIM = 128
NUM_CORES = 2
NUM_SUBCORES = 16
NUM_WORKERS = NUM_CORES * NUM_SUBCORES
STRIPE = 128
NBUF = 5


def _emb_body(idx_hbm, table_hbm, out_hbm, idx_v, rows_v, gsem, osem):
    n_groups = idx_hbm.shape[1]
    n_rot = n_groups // NBUF
    wid = lax.axis_index("s") * NUM_CORES + lax.axis_index("c")
    b0 = wid * STRIPE
    pltpu.sync_copy(idx_hbm.at[wid], idx_v)

    def gstart(g, b):
        pltpu.async_copy(table_hbm.at[idx_v.at[g]], rows_v.at[b], gsem.at[b])

    def gwait(g, b):
        pltpu.make_async_copy(table_hbm.at[idx_v.at[g]], rows_v.at[b],
                              gsem.at[b]).wait()

    def wstart(g, b):
        pltpu.async_copy(rows_v.at[b], out_hbm.at[g, pl.ds(b0, STRIPE)],
                         osem.at[b])

    def wwait(g, b):
        pltpu.make_async_copy(rows_v.at[b], out_hbm.at[g, pl.ds(b0, STRIPE)],
                              osem.at[b]).wait()

    for b in range(NBUF):
        gstart(b, b)

    def body(r, carry):
        g0 = r * NBUF
        for b in range(NBUF):
            gwait(g0 + b, b)
            wstart(g0 + b, b)
        for b in range(NBUF):
            wwait(g0 + b, b)
            gstart(g0 + NBUF + b, b)
        return carry

    lax.fori_loop(0, n_rot - 1, body, 0)

    g0 = (n_rot - 1) * NBUF
    for b in range(NBUF):
        gwait(g0 + b, b)
        wstart(g0 + b, b)
    for b in range(NBUF):
        wwait(g0 + b, b)


def kernel(input_ids, embedding_weight):
    batch, seq = input_ids.shape
    assert batch % (NUM_WORKERS * STRIPE) == 0 or batch == NUM_WORKERS * STRIPE
    assert seq % NBUF == 0

    idx = (input_ids.astype(jnp.int32)
           .reshape(NUM_WORKERS, STRIPE, seq)
           .transpose(0, 2, 1))
    mesh = plsc.VectorSubcoreMesh(core_axis_name="c", subcore_axis_name="s")
    out = pl.kernel(
        _emb_body,
        out_type=jax.ShapeDtypeStruct((seq, batch, DIM), jnp.float32),
        mesh=mesh,
        scratch_types=[
            pltpu.VMEM((seq, STRIPE), jnp.int32),
            pltpu.VMEM((NBUF, STRIPE, DIM), jnp.float32),
            pltpu.SemaphoreType.DMA((NBUF,)),
            pltpu.SemaphoreType.DMA((NBUF,)),
        ],
    )(idx, embedding_weight)
    return out.transpose(1, 0, 2)


# speedup vs baseline: 10.5451x; 1.0179x over previous
import jax
import jax.numpy as jnp
from jax import lax
from jax.experimental import pallas as pl
from jax.experimental.pallas import tpu as pltpu
from jax.experimental.pallas import tpu_sc as plsc

DIM = 128
NUM_CORES = 2
NUM_SUBCORES = 16
NUM_WORKERS = NUM_CORES * NUM_SUBCORES
STRIPE = 128
CHUNK = 64
NBUF = 10


def _emb_body(idx_hbm, table_hbm, out_hbm, idx_v, rows_v, gsem, osem):
    n_groups = idx_hbm.shape[1] * (STRIPE // CHUNK)
    n_rot = n_groups // NBUF
    wid = lax.axis_index("s") * NUM_CORES + lax.axis_index("c")
    b0 = wid * STRIPE
    pltpu.sync_copy(idx_hbm.at[wid], idx_v)

    def gstart(g, b):
        s, h = g // 2, g % 2
        pltpu.async_copy(table_hbm.at[idx_v.at[s, pl.ds(h * CHUNK, CHUNK)]],
                         rows_v.at[b], gsem.at[b])

    def gwait(g, b):
        s, h = g // 2, g % 2
        pltpu.make_async_copy(
            table_hbm.at[idx_v.at[s, pl.ds(h * CHUNK, CHUNK)]],
            rows_v.at[b], gsem.at[b]).wait()

    def wstart(g, b):
        s, h = g // 2, g % 2
        pltpu.async_copy(rows_v.at[b],
                         out_hbm.at[s, pl.ds(b0 + h * CHUNK, CHUNK)],
                         osem.at[b])

    def wwait(g, b):
        s, h = g // 2, g % 2
        pltpu.make_async_copy(rows_v.at[b],
                              out_hbm.at[s, pl.ds(b0 + h * CHUNK, CHUNK)],
                              osem.at[b]).wait()

    for b in range(NBUF):
        gstart(b, b)

    def body(r, carry):
        g0 = r * NBUF
        for b in range(NBUF):
            gwait(g0 + b, b)
            wstart(g0 + b, b)
        for b in range(NBUF):
            wwait(g0 + b, b)
            gstart(g0 + NBUF + b, b)
        return carry

    lax.fori_loop(0, n_rot - 1, body, 0)

    g0 = (n_rot - 1) * NBUF
    for b in range(NBUF):
        gwait(g0 + b, b)
        wstart(g0 + b, b)
    for b in range(NBUF):
        wwait(g0 + b, b)


def kernel(input_ids, embedding_weight):
    batch, seq = input_ids.shape
    assert batch == NUM_WORKERS * STRIPE
    assert (seq * STRIPE // CHUNK) % NBUF == 0

    idx = (input_ids.astype(jnp.int32)
           .reshape(NUM_WORKERS, STRIPE, seq)
           .transpose(0, 2, 1))
    mesh = plsc.VectorSubcoreMesh(core_axis_name="c", subcore_axis_name="s")
    out = pl.kernel(
        _emb_body,
        out_type=jax.ShapeDtypeStruct((seq, batch, DIM), jnp.float32),
        mesh=mesh,
        scratch_types=[
            pltpu.VMEM((seq, STRIPE), jnp.int32),
            pltpu.VMEM((NBUF, CHUNK, DIM), jnp.float32),
            pltpu.SemaphoreType.DMA((NBUF,)),
            pltpu.SemaphoreType.DMA((NBUF,)),
        ],
    )(idx, embedding_weight)
    return out.transpose(1, 0, 2)
